# Initial kernel scaffold; baseline (speedup 1.0000x reference)
import jax
import jax.numpy as jnp
from jax.experimental import pallas as pl


def kernel(features, edge_index, adj_values, weight):
    raise NotImplementedError("write your pallas kernel here")


# speedup vs baseline: 3.5280x; 3.5280x over previous
import functools

import jax
import jax.numpy as jnp
from jax import lax
from jax.experimental import pallas as pl
from jax.experimental.pallas import tpu as pltpu
from jax.experimental.pallas import tpu_sc as plsc

_N = 10000
_E = 160000
_DIN = 256
_DOUT = 256
_H = 128
_NCORE = 2
_NSUB = 16
_CH = 128
_NCH = _E // _CH
_NWR = 10
_RPB = _N // _NWR


def _sc_aggregate(fcat, row2, col2, adj2, zeros):
    mesh = plsc.VectorSubcoreMesh(core_axis_name="c", subcore_axis_name="s")

    @functools.partial(
        pl.kernel,
        out_type=jax.ShapeDtypeStruct((_NCORE * _N, _H), jnp.float32),
        mesh=mesh,
        scratch_types=[
            pltpu.VMEM((1, _CH), jnp.int32),
            pltpu.VMEM((1, _CH), jnp.int32),
            pltpu.VMEM((1, _CH), jnp.float32),
            pltpu.VMEM((_CH, _H), jnp.float32),
            pltpu.VMEM_SHARED((_N, _H), jnp.float32),
        ],
    )
    def agg(fcat_hbm, row_hbm, col_hbm, adj_hbm, zeros_hbm, out_hbm,
            ridx_v, cidx_v, adj_v, rows_v, acc_sh):
        c = lax.axis_index("c")
        s = lax.axis_index("s")

        @pl.when(s < _NWR)
        def _zero():
            pltpu.sync_copy(zeros_hbm, acc_sh.at[pl.ds(s * _RPB, _RPB)])
        plsc.subcore_barrier()

        @pl.loop(s, _NCH, step=_NSUB)
        def _chunk(k):
            pltpu.sync_copy(col_hbm.at[k], cidx_v)
            pltpu.sync_copy(row_hbm.at[k], ridx_v)
            pltpu.sync_copy(adj_hbm.at[k], adj_v)

            @pl.loop(0, _CH, step=16)
            def _gi(i):
                sl = pl.ds(i, 16)
                cidx_v[0, sl] = cidx_v[0, sl] * 2 + c

            pltpu.sync_copy(fcat_hbm.at[cidx_v.at[0]], rows_v)

            @pl.loop(0, _CH, step=16)
            def _scale(g):
                av = adj_v[0, pl.ds(g, 16)]
                for l in range(16):
                    a = av[l]
                    for j in range(_H // 16):
                        sl = pl.ds(j * 16, 16)
                        rows_v[g + l, sl] = rows_v[g + l, sl] * a

            pltpu.sync_copy(rows_v, acc_sh.at[ridx_v.at[0]], add=True)

        plsc.subcore_barrier()

        @pl.when(s < _NWR)
        def _writeback():
            pltpu.sync_copy(acc_sh.at[pl.ds(s * _RPB, _RPB)],
                            out_hbm.at[pl.ds(c * _N + s * _RPB, _RPB)])

    return agg(fcat, row2, col2, adj2, zeros)


def _tc_matmul_lrelu(agg2, w2):
    bm = 1000

    def body(a_ref, w_ref, o_ref):
        acc = jnp.dot(a_ref[0], w_ref[0], preferred_element_type=jnp.float32)
        acc = acc + jnp.dot(a_ref[1], w_ref[1],
                            preferred_element_type=jnp.float32)
        o_ref[...] = jnp.where(acc >= 0.0, acc, 0.2 * acc)

    return pl.pallas_call(
        body,
        grid=(_N // bm,),
        in_specs=[
            pl.BlockSpec((2, bm, _H), lambda i: (0, i, 0)),
            pl.BlockSpec((2, _H, _DOUT), lambda i: (0, 0, 0)),
        ],
        out_specs=pl.BlockSpec((bm, _DOUT), lambda i: (i, 0)),
        out_shape=jax.ShapeDtypeStruct((_N, _DOUT), jnp.float32),
    )(agg2, w2)


def kernel(features, edge_index, adj_values, weight):
    row2 = edge_index[0].astype(jnp.int32).reshape(_NCH, 1, _CH)
    col2 = edge_index[1].astype(jnp.int32).reshape(_NCH, 1, _CH)
    adj2 = adj_values.reshape(_NCH, 1, _CH)
    fcat = features.reshape(_N * _NCORE, _H)
    zeros = jnp.zeros((_RPB, _H), jnp.float32)
    agg = _sc_aggregate(fcat, row2, col2, adj2, zeros)
    agg2 = agg.reshape(_NCORE, _N, _H)
    w2 = weight.reshape(_NCORE, _H, _DOUT)
    return _tc_matmul_lrelu(agg2, w2)


# speedup vs baseline: 3.7285x; 1.0569x over previous
import dataclasses
import functools

import jax
import jax.numpy as jnp
from jax import lax
from jax.experimental import pallas as pl
from jax.experimental.pallas import tpu as pltpu
from jax.experimental.pallas import tpu_sc as plsc

_N = 10000
_E = 160000
_DIN = 256
_DOUT = 256
_H = 128
_NCORE = 2
_NSUB = 16
_CH = 128
_EP = 163840
_NCHP = _EP // _CH
_CPS = _NCHP // _NSUB
_NWR = 10
_RPB = _N // _NWR


def _sc_aggregate(fcat, colaj, rowp, zeros):
    mesh = plsc.VectorSubcoreMesh(core_axis_name="c", subcore_axis_name="s")

    scratch = (
        [pltpu.VMEM((2, _CH), jnp.int32)] * 4
        + [pltpu.VMEM((1, _CH), jnp.int32)] * 4
        + [pltpu.VMEM((1, _CH), jnp.int32)] * 2
        + [pltpu.VMEM((_CH, _H), jnp.float32)] * 2
        + [pltpu.VMEM_SHARED((_N, _H), jnp.float32)]
        + [pltpu.SemaphoreType.DMA] * 12
    )

    cp = pltpu.CompilerParams()
    if "needs_layout_passes" in pltpu.CompilerParams.__dataclass_fields__:
        cp = dataclasses.replace(cp, needs_layout_passes=False)

    @functools.partial(
        pl.kernel,
        out_type=jax.ShapeDtypeStruct((_NCORE * _N, _H), jnp.float32),
        mesh=mesh,
        scratch_types=scratch,
        compiler_params=cp,
    )
    def agg(fcat_hbm, colaj_hbm, rowp_hbm, zeros_hbm, out_hbm, *sc):
        ca = list(sc[0:4])
        ri = list(sc[4:8])
        gx = list(sc[8:10])
        rw = list(sc[10:12])
        acc_sh = sc[12]
        ca_sem = list(sc[13:17])
        ri_sem = list(sc[17:21])
        g_sem = list(sc[21:23])
        sc_sem = list(sc[23:25])

        c = lax.axis_index("c")
        s = lax.axis_index("s")
        w0 = s * _CPS

        @pl.when(s < _NWR)
        def _zero():
            pltpu.sync_copy(zeros_hbm, acc_sh.at[pl.ds(s * _RPB, _RPB)])
        plsc.subcore_barrier()

        def load_meta(t, q):
            pltpu.async_copy(colaj_hbm.at[t], ca[q], ca_sem[q])
            pltpu.async_copy(rowp_hbm.at[t], ri[q], ri_sem[q])

        def wait_meta(t, q):
            pltpu.make_async_copy(colaj_hbm.at[t], ca[q], ca_sem[q]).wait()
            pltpu.make_async_copy(rowp_hbm.at[t], ri[q], ri_sem[q]).wait()

        def compute_gx(qm, qg):
            for i in range(0, _CH, 16):
                sl = pl.ds(i, 16)
                gx[qg][0, sl] = ca[qm][0, sl] + c

        def start_gather(qg, b):
            pltpu.async_copy(fcat_hbm.at[gx[qg].at[0]], rw[b], g_sem[b])

        def wait_gather(qg, b):
            pltpu.make_async_copy(
                fcat_hbm.at[gx[qg].at[0]], rw[b], g_sem[b]).wait()

        def scale(b, qm):
            @pl.loop(0, _CH, step=16)
            def _scale(g):
                av = plsc.bitcast(ca[qm][1, pl.ds(g, 16)], jnp.float32)
                for l in range(16):
                    a = av[l]
                    for jj in range(_H // 16):
                        sl = pl.ds(jj * 16, 16)
                        rw[b][g + l, sl] = rw[b][g + l, sl] * a

        def wait_scatter(b, qm):
            pltpu.make_async_copy(
                rw[b], acc_sh.at[ri[qm].at[0]], sc_sem[b]).wait()

        load_meta(w0, 0)
        load_meta(w0 + 1, 1)
        wait_meta(w0, 0)
        compute_gx(0, 0)
        start_gather(0, 0)

        @pl.loop(0, _CPS, step=4)
        def _body(tb):
            for o in range(4):
                t = tb + o
                b = o % 2
                nb = 1 - b
                q1 = (o + 1) % 4
                q2 = (o + 2) % 4

                @pl.when(t + 1 < _CPS)
                def _p1():
                    wait_meta(w0 + t + 1, q1)
                    compute_gx(q1, nb)

                    @pl.when(t > 0)
                    def _p2():
                        wait_scatter(nb, q1)
                    start_gather(nb, nb)

                @pl.when(t + 2 < _CPS)
                def _p4():
                    load_meta(w0 + t + 2, q2)

                wait_gather(b, b)
                scale(b, o)
                h = pltpu.async_copy(
                    rw[b], acc_sh.at[ri[o].at[0]], sc_sem[b], add=True)

                @pl.when(t >= _CPS - 2)
                def _drain():
                    h.wait()

        plsc.subcore_barrier()

        @pl.when(s < _NWR)
        def _writeback():
            pltpu.sync_copy(acc_sh.at[pl.ds(s * _RPB, _RPB)],
                            out_hbm.at[pl.ds(c * _N + s * _RPB, _RPB)])

    return agg(fcat, colaj, rowp, zeros)


def _tc_matmul_lrelu(agg2, w2):
    bm = 1000

    def body(a_ref, w_ref, o_ref):
        acc = jnp.dot(a_ref[0], w_ref[0], preferred_element_type=jnp.float32)
        acc = acc + jnp.dot(a_ref[1], w_ref[1],
                            preferred_element_type=jnp.float32)
        o_ref[...] = jnp.where(acc >= 0.0, acc, 0.2 * acc)

    return pl.pallas_call(
        body,
        grid=(_N // bm,),
        in_specs=[
            pl.BlockSpec((2, bm, _H), lambda i: (0, i, 0)),
            pl.BlockSpec((2, _H, _DOUT), lambda i: (0, 0, 0)),
        ],
        out_specs=pl.BlockSpec((bm, _DOUT), lambda i: (i, 0)),
        out_shape=jax.ShapeDtypeStruct((_N, _DOUT), jnp.float32),
    )(agg2, w2)


def kernel(features, edge_index, adj_values, weight):
    pad = _EP - _E
    row_p = jnp.concatenate(
        [edge_index[0].astype(jnp.int32), jnp.zeros((pad,), jnp.int32)])
    col2_p = jnp.concatenate(
        [edge_index[1].astype(jnp.int32) * 2, jnp.zeros((pad,), jnp.int32)])
    adj_p = jnp.concatenate([adj_values, jnp.zeros((pad,), jnp.float32)])
    colaj = jnp.stack(
        [col2_p.reshape(_NCHP, _CH),
         lax.bitcast_convert_type(adj_p, jnp.int32).reshape(_NCHP, _CH)],
        axis=1)
    rowp = row_p.reshape(_NCHP, 1, _CH)
    fcat = features.reshape(_N * _NCORE, _H)
    zeros = jnp.zeros((_RPB, _H), jnp.float32)
    agg = _sc_aggregate(fcat, colaj, rowp, zeros)
    agg2 = agg.reshape(_NCORE, _N, _H)
    w2 = weight.reshape(_NCORE, _H, _DOUT)
    return _tc_matmul_lrelu(agg2, w2)
